# Initial kernel scaffold; baseline (speedup 1.0000x reference)
import jax
import jax.numpy as jnp
from jax.experimental import pallas as pl


def kernel(x_s, x_t, edge_index, edge_attr, Wf0, Wf1, Wf2, W1, b1, W2, b2):
    raise NotImplementedError("write your pallas kernel here")


# speedup vs baseline: 23.3201x; 23.3201x over previous
import functools

import jax
import jax.numpy as jnp
from jax import lax
from jax.experimental import pallas as pl
from jax.experimental.pallas import tpu as pltpu
from jax.experimental.pallas import tpu_sc as plsc

N = 50000
E = 1600000
S = 8
D = 8
F = 4
HID = 16
TW = S + D

NC = 2
NS = 16
NW = NC * NS
EPW = E // NW
CH = 2000
NCHUNK = EPW // CH

MLP_B = 16000
MLP_R = MLP_B // 8
NB = 2000


@functools.cache
def _sc_gather_kernel():
    mesh = plsc.VectorSubcoreMesh(core_axis_name="c", subcore_axis_name="s")

    @functools.partial(
        pl.kernel,
        mesh=mesh,
        out_type=[
            jax.ShapeDtypeStruct((E, TW), jnp.float32),
            jax.ShapeDtypeStruct((E, TW), jnp.float32),
        ],
        scratch_types=[
            pltpu.VMEM((CH,), jnp.int32),
            pltpu.VMEM((CH, TW), jnp.float32),
            pltpu.VMEM((CH,), jnp.int32),
            pltpu.VMEM((CH, TW), jnp.float32),
            pltpu.SemaphoreType.DMA,
            pltpu.SemaphoreType.DMA,
        ],
        compiler_params=pltpu.CompilerParams(use_tc_tiling_on_sc=False),
    )
    def _sc_gather(t_hbm, row_hbm, col_hbm, gr_hbm, gc_hbm,
                   idx_r, rows_r, idx_c, rows_c, sem_r, sem_c):
        cid = lax.axis_index("c")
        sid = lax.axis_index("s")
        wid = cid * NS + sid
        base = wid * EPW
        for j in range(NCHUNK):
            off = base + j * CH
            pltpu.sync_copy(row_hbm.at[pl.ds(off, CH)], idx_r)
            pltpu.sync_copy(col_hbm.at[pl.ds(off, CH)], idx_c)
            a = pltpu.async_copy(t_hbm.at[idx_r], rows_r, sem_r)
            b = pltpu.async_copy(t_hbm.at[idx_c], rows_c, sem_c)
            a.wait()
            b.wait()
            pltpu.sync_copy(rows_r, gr_hbm.at[pl.ds(off, CH)])
            pltpu.sync_copy(rows_c, gc_hbm.at[pl.ds(off, CH)])

    return _sc_gather


ACC_CH = N // NS


@functools.cache
def _sc_scatter_kernel():
    mesh = plsc.VectorSubcoreMesh(core_axis_name="c", subcore_axis_name="s")

    @functools.partial(
        pl.kernel,
        mesh=mesh,
        out_type=jax.ShapeDtypeStruct((NC, N, D), jnp.float32),
        scratch_types=[
            pltpu.VMEM((CH, D), jnp.float32),
            pltpu.VMEM((CH,), jnp.int32),
            pltpu.VMEM((ACC_CH, D), jnp.float32),
            pltpu.VMEM_SHARED((N, D), jnp.float32),
        ],
        compiler_params=pltpu.CompilerParams(use_tc_tiling_on_sc=False),
    )
    def _sc_scatter(shift_hbm, col_hbm, zeros_hbm, part_hbm,
                    sh_v, idx_v, cbuf, acc):
        cid = lax.axis_index("c")
        sid = lax.axis_index("s")
        @pl.when(sid == 0)
        def _():
            pltpu.sync_copy(zeros_hbm, acc)
        plsc.subcore_barrier()
        base = (cid * NS + sid) * EPW
        for j in range(NCHUNK):
            off = base + j * CH
            pltpu.sync_copy(shift_hbm.at[pl.ds(off, CH), pl.ds(S, D)], sh_v)
            pltpu.sync_copy(col_hbm.at[pl.ds(off, CH)], idx_v)
            pltpu.sync_copy(sh_v, acc.at[idx_v], add=True)
        plsc.subcore_barrier()
        r0 = sid * ACC_CH
        pltpu.sync_copy(acc.at[pl.ds(r0, ACC_CH)], cbuf)
        pltpu.sync_copy(cbuf, part_hbm.at[cid].at[pl.ds(r0, ACC_CH)])

    return _sc_scatter


def _mlp_body(grp_ref, gcp_ref, eav_ref, bdr_ref, bdc_ref, me_ref,
              b1t_ref, bdw2_ref, b2t_ref, bdon_ref, bdms_ref, out_ref):
    g_r = grp_ref[...]
    g_c = gcp_ref[...]
    dot = functools.partial(jnp.dot, preferred_element_type=jnp.float32)
    h = (dot(g_r, bdr_ref[...]) + dot(g_c, bdc_ref[...])
         + dot(eav_ref[...], me_ref[...]) + b1t_ref[...])
    h = jnp.maximum(h, 0.0)
    w = dot(h, bdw2_ref[...]) + b2t_ref[...]
    n2 = dot(w * w, bdon_ref[...])
    inv = jnp.where(n2 > 0, lax.rsqrt(n2), 0.0)
    sr = dot(g_r, bdms_ref[...])
    sc = dot(g_c, bdms_ref[...])
    m = jnp.logical_or(sr != 0, sc != 0)
    out_ref[...] = (g_c - g_r) * (w * inv) * m.astype(jnp.float32)


def _mlp_call(grp, gcp, eav, bdr, bdc, me, b1t, bdw2, b2t, bdon, bdms):
    grid = (E // MLP_B,)
    return pl.pallas_call(
        _mlp_body,
        grid=grid,
        in_specs=[
            pl.BlockSpec((MLP_R, 128), lambda i: (i, 0)),
            pl.BlockSpec((MLP_R, 128), lambda i: (i, 0)),
            pl.BlockSpec((MLP_R, 32), lambda i: (i, 0)),
            pl.BlockSpec((128, 128), lambda i: (0, 0)),
            pl.BlockSpec((128, 128), lambda i: (0, 0)),
            pl.BlockSpec((32, 128), lambda i: (0, 0)),
            pl.BlockSpec((1, 128), lambda i: (0, 0)),
            pl.BlockSpec((128, 128), lambda i: (0, 0)),
            pl.BlockSpec((1, 128), lambda i: (0, 0)),
            pl.BlockSpec((128, 128), lambda i: (0, 0)),
            pl.BlockSpec((128, 128), lambda i: (0, 0)),
        ],
        out_specs=pl.BlockSpec((MLP_R, 128), lambda i: (i, 0)),
        out_shape=jax.ShapeDtypeStruct((E // 8, 128), jnp.float32),
    )(grp, gcp, eav, bdr, bdc, me, b1t, bdw2, b2t, bdon, bdms)


def _init_body(xt_ref, xs_ref, wf_ref, out_ref, t_ref):
    out = jnp.dot(xt_ref[...], wf_ref[...], preferred_element_type=jnp.float32)
    out_ref[...] = out
    t_ref[...] = jnp.concatenate([xs_ref[...], out], axis=1)


def _init_call(x_t, x_s, wf0):
    grid = (N // NB,)
    return pl.pallas_call(
        _init_body,
        grid=grid,
        in_specs=[
            pl.BlockSpec((NB, D), lambda i: (i, 0)),
            pl.BlockSpec((NB, S), lambda i: (i, 0)),
            pl.BlockSpec((D, D), lambda i: (0, 0)),
        ],
        out_specs=[
            pl.BlockSpec((NB, D), lambda i: (i, 0)),
            pl.BlockSpec((NB, TW), lambda i: (i, 0)),
        ],
        out_shape=[
            jax.ShapeDtypeStruct((N, D), jnp.float32),
            jax.ShapeDtypeStruct((N, TW), jnp.float32),
        ],
    )(x_t, x_s, wf0)


def _update_body(out_ref, p0_ref, p1_ref, wf_ref, xs_ref, new_ref, t_ref):
    scat = p0_ref[...] + p1_ref[...]
    new = out_ref[...] + jnp.dot(scat, wf_ref[...],
                                 preferred_element_type=jnp.float32)
    new_ref[...] = new
    t_ref[...] = jnp.concatenate([xs_ref[...], new], axis=1)


def _update_call(out, p0, p1, wf, x_s):
    grid = (N // NB,)
    return pl.pallas_call(
        _update_body,
        grid=grid,
        in_specs=[
            pl.BlockSpec((NB, D), lambda i: (i, 0)),
            pl.BlockSpec((NB, D), lambda i: (i, 0)),
            pl.BlockSpec((NB, D), lambda i: (i, 0)),
            pl.BlockSpec((D, D), lambda i: (0, 0)),
            pl.BlockSpec((NB, S), lambda i: (i, 0)),
        ],
        out_specs=[
            pl.BlockSpec((NB, D), lambda i: (i, 0)),
            pl.BlockSpec((NB, TW), lambda i: (i, 0)),
        ],
        out_shape=[
            jax.ShapeDtypeStruct((N, D), jnp.float32),
            jax.ShapeDtypeStruct((N, TW), jnp.float32),
        ],
    )(out, p0, p1, wf, x_s)


def kernel(x_s, x_t, edge_index, edge_attr, Wf0, Wf1, Wf2, W1, b1, W2, b2):
    f32 = jnp.float32
    row = edge_index[0]
    col = edge_index[1]
    eye8 = jnp.eye(8, dtype=f32)
    w1r = jnp.concatenate([W1[0:S], W1[2 * S:2 * S + D]], axis=0)
    w1c = jnp.concatenate([W1[S:2 * S], W1[2 * S + D:2 * S + 2 * D]], axis=0)
    w1e = W1[2 * S + 2 * D:]
    bdr = jnp.kron(eye8, w1r)
    bdc = jnp.kron(eye8, w1c)
    me = jnp.kron(eye8, w1e)
    b1t = jnp.tile(b1, 8).reshape(1, 128)
    w2pad = jnp.concatenate([jnp.zeros((HID, S), f32), W2], axis=1)
    bdw2 = jnp.kron(eye8, w2pad)
    b2t = jnp.tile(jnp.concatenate([jnp.zeros((S,), f32), b2]), 8)
    b2t = b2t.reshape(1, 128)
    bdon = jnp.kron(eye8, jnp.ones((TW, TW), f32))
    mout = jnp.concatenate(
        [jnp.zeros((S, TW), f32), jnp.ones((D, TW), f32)], axis=0)
    bdms = jnp.kron(eye8, mout)
    eav = edge_attr.reshape(E // 8, 8 * F)
    zeros_nt = jnp.zeros((N, D), dtype=f32)

    out, t = _init_call(x_t, x_s, Wf0)
    for wf in (Wf1, Wf2):
        gr, gc = _sc_gather_kernel()(t, row, col)
        grp = gr.reshape(E // 8, 128)
        gcp = gc.reshape(E // 8, 128)
        shiftp = _mlp_call(grp, gcp, eav, bdr, bdc, me, b1t, bdw2, b2t,
                           bdon, bdms)
        shift = shiftp.reshape(E, TW)
        parts = _sc_scatter_kernel()(shift, col, zeros_nt)
        out, t = _update_call(out, parts[0], parts[1], wf, x_s)
    return out


# speedup vs baseline: 44.6891x; 1.9163x over previous
import functools

import jax
import jax.numpy as jnp
from jax import lax
from jax.experimental import pallas as pl
from jax.experimental.pallas import tpu as pltpu
from jax.experimental.pallas import tpu_sc as plsc

N = 50000
E = 1600000
S = 8
D = 8
F = 4
HID = 16
TW = S + D

NC = 2
NS = 16
NW = NC * NS
EPW = E // NW
GCH = 1000
GNCH = EPW // GCH
CH = 1000
NCHUNK = EPW // CH

MLP_B = 16000
MLP_R = MLP_B // 8
NB = 2000


@functools.cache
def _sc_gather_kernel():
    mesh = plsc.VectorSubcoreMesh(core_axis_name="c", subcore_axis_name="s")

    @functools.partial(
        pl.kernel,
        mesh=mesh,
        out_type=[
            jax.ShapeDtypeStruct((E, TW), jnp.float32),
            jax.ShapeDtypeStruct((E, TW), jnp.float32),
        ],
        scratch_types=[
            [pltpu.VMEM((GCH,), jnp.int32)] * 2,
            [pltpu.VMEM((GCH, TW), jnp.float32)] * 2,
            [pltpu.VMEM((GCH,), jnp.int32)] * 2,
            [pltpu.VMEM((GCH, TW), jnp.float32)] * 2,
            [pltpu.SemaphoreType.DMA] * 2,
            [pltpu.SemaphoreType.DMA] * 2,
            [pltpu.SemaphoreType.DMA] * 2,
            [pltpu.SemaphoreType.DMA] * 2,
        ],
        compiler_params=pltpu.CompilerParams(use_tc_tiling_on_sc=False),
    )
    def _sc_gather(t_hbm, ei_hbm, gr_hbm, gc_hbm,
                   idx_r, rows_r, idx_c, rows_c,
                   sem_ix, sem_g, sem_wr, sem_wc):
        cid = lax.axis_index("c")
        sid = lax.axis_index("s")
        wid = cid * NS + sid
        base = wid * EPW

        def load_idx(j, p):
            off = base + j * GCH
            pltpu.async_copy(ei_hbm.at[0, pl.ds(off, GCH)], idx_r[p],
                             sem_ix[p])
            pltpu.async_copy(ei_hbm.at[1, pl.ds(off, GCH)], idx_c[p],
                             sem_ix[p])

        def wait_wb(j, p):
            off = base + j * GCH
            pltpu.make_async_copy(rows_r[p], gr_hbm.at[pl.ds(off, GCH)],
                                  sem_wr[p]).wait()
            pltpu.make_async_copy(rows_c[p], gc_hbm.at[pl.ds(off, GCH)],
                                  sem_wc[p]).wait()

        def chunk(j, p, first, last):
            off = base + j * GCH
            pltpu.make_async_copy(ei_hbm.at[0, pl.ds(off, GCH)], idx_r[p],
                                  sem_ix[p]).wait()
            pltpu.make_async_copy(ei_hbm.at[1, pl.ds(off, GCH)], idx_c[p],
                                  sem_ix[p]).wait()
            if not first:
                wait_wb(j - 2, p)
            a = pltpu.async_copy(t_hbm.at[idx_r[p]], rows_r[p], sem_g[p])
            b = pltpu.async_copy(t_hbm.at[idx_c[p]], rows_c[p], sem_g[p])
            a.wait()
            b.wait()
            pltpu.async_copy(rows_r[p], gr_hbm.at[pl.ds(off, GCH)],
                             sem_wr[p])
            pltpu.async_copy(rows_c[p], gc_hbm.at[pl.ds(off, GCH)],
                             sem_wc[p])
            if not last:
                load_idx(j + 2, p)

        load_idx(0, 0)
        load_idx(1, 1)
        chunk(0, 0, True, False)
        chunk(1, 1, True, False)

        @pl.loop(2, GNCH - 2, step=2)
        def _(g):
            chunk(g, 0, False, False)
            chunk(g + 1, 1, False, False)

        chunk(GNCH - 2, 0, False, True)
        chunk(GNCH - 1, 1, False, True)
        wait_wb(GNCH - 2, 0)
        wait_wb(GNCH - 1, 1)

    return _sc_gather


ACC_CH = N // NS


@functools.cache
def _sc_scatter_kernel():
    mesh = plsc.VectorSubcoreMesh(core_axis_name="c", subcore_axis_name="s")

    @functools.partial(
        pl.kernel,
        mesh=mesh,
        out_type=[
            jax.ShapeDtypeStruct((N, D), jnp.float32),
            jax.ShapeDtypeStruct((N, D), jnp.float32),
        ],
        scratch_types=[
            [pltpu.VMEM((CH, D), jnp.float32)] * 2,
            [pltpu.VMEM((CH,), jnp.int32)] * 2,
            [pltpu.SemaphoreType.DMA] * 2,
            pltpu.VMEM((ACC_CH, D), jnp.float32),
            pltpu.VMEM_SHARED((N, D), jnp.float32),
        ],
        compiler_params=pltpu.CompilerParams(use_tc_tiling_on_sc=False),
    )
    def _sc_scatter(shift_hbm, ei_hbm, zeros_hbm, p0_hbm, p1_hbm,
                    sh_v, idx_v, sem_ld, cbuf, acc):
        cid = lax.axis_index("c")
        sid = lax.axis_index("s")
        @pl.when(sid == 0)
        def _():
            pltpu.sync_copy(zeros_hbm, acc)
        plsc.subcore_barrier()
        base = (cid * NS + sid) * EPW

        def load(j, p):
            off = base + j * CH
            pltpu.async_copy(shift_hbm.at[pl.ds(off, CH), pl.ds(S, D)],
                             sh_v[p], sem_ld[p])
            pltpu.async_copy(ei_hbm.at[1, pl.ds(off, CH)], idx_v[p],
                             sem_ld[p])

        def chunk(j, p, last):
            off = base + j * CH
            pltpu.make_async_copy(shift_hbm.at[pl.ds(off, CH), pl.ds(S, D)],
                                  sh_v[p], sem_ld[p]).wait()
            pltpu.make_async_copy(ei_hbm.at[1, pl.ds(off, CH)], idx_v[p],
                                  sem_ld[p]).wait()
            pltpu.sync_copy(sh_v[p], acc.at[idx_v[p]], add=True)
            if not last:
                load(j + 2, p)

        load(0, 0)
        load(1, 1)

        @pl.loop(0, NCHUNK - 2, step=2)
        def _(g):
            chunk(g, 0, False)
            chunk(g + 1, 1, False)

        chunk(NCHUNK - 2, 0, True)
        chunk(NCHUNK - 1, 1, True)
        plsc.subcore_barrier()
        r0 = sid * ACC_CH
        pltpu.sync_copy(acc.at[pl.ds(r0, ACC_CH)], cbuf)
        @pl.when(cid == 0)
        def _():
            pltpu.sync_copy(cbuf, p0_hbm.at[pl.ds(r0, ACC_CH)])
        @pl.when(cid == 1)
        def _():
            pltpu.sync_copy(cbuf, p1_hbm.at[pl.ds(r0, ACC_CH)])

    return _sc_scatter


def _mlp_body(grp_ref, gcp_ref, ea0_ref, ea1_ref, ea2_ref, ea3_ref,
              bdr_ref, bdc_ref, se0_ref, se1_ref, se2_ref, se3_ref,
              b1t_ref, bdw2_ref, b2t_ref, bdon_ref, bdms_ref,
              out_ref):
    g_r = grp_ref[...]
    g_c = gcp_ref[...]
    dot = functools.partial(jnp.dot, preferred_element_type=jnp.float32)
    h = (dot(g_r, bdr_ref[...]) + dot(g_c, bdc_ref[...])
         + dot(ea0_ref[...], se0_ref[...]) + dot(ea1_ref[...], se1_ref[...])
         + dot(ea2_ref[...], se2_ref[...]) + dot(ea3_ref[...], se3_ref[...])
         + b1t_ref[...])
    h = jnp.maximum(h, 0.0)
    w = dot(h, bdw2_ref[...]) + b2t_ref[...]
    n2 = dot(w * w, bdon_ref[...])
    inv = jnp.where(n2 > 0, lax.rsqrt(n2), 0.0)
    m = jnp.logical_or(dot(g_r, bdms_ref[...]) != 0,
                       dot(g_c, bdms_ref[...]) != 0)
    out_ref[...] = (g_c - g_r) * (w * inv) * m.astype(jnp.float32)


def _mlp_call(grp, gcp, eacols, bdr, bdc, sels, b1t, bdw2, b2t, bdon, bdms):
    grid = (E // MLP_B,)
    edge_spec = pl.BlockSpec((MLP_R, 128), lambda i: (i, 0))
    col_spec = pl.BlockSpec((MLP_R, 8), lambda i: (i, 0))
    full = lambda shape: pl.BlockSpec(shape, lambda i: (0, 0))
    return pl.pallas_call(
        _mlp_body,
        grid=grid,
        in_specs=[
            edge_spec, edge_spec,
            col_spec, col_spec, col_spec, col_spec,
            full((128, 128)), full((128, 128)),
            full((8, 128)), full((8, 128)), full((8, 128)), full((8, 128)),
            full((1, 128)), full((128, 128)), full((1, 128)),
            full((128, 128)), full((128, 128)),
        ],
        out_specs=pl.BlockSpec((MLP_R, 128), lambda i: (i, 0)),
        out_shape=jax.ShapeDtypeStruct((E // 8, 128), jnp.float32),
    )(grp, gcp, *eacols, bdr, bdc, *sels, b1t, bdw2, b2t, bdon, bdms)


def _init_body(xt_ref, xs_ref, wf_ref, out_ref, t_ref):
    out = jnp.dot(xt_ref[...], wf_ref[...], preferred_element_type=jnp.float32)
    out_ref[...] = out
    t_ref[...] = jnp.concatenate([xs_ref[...], out], axis=1)


def _init_call(x_t, x_s, wf0):
    grid = (N // NB,)
    return pl.pallas_call(
        _init_body,
        grid=grid,
        in_specs=[
            pl.BlockSpec((NB, D), lambda i: (i, 0)),
            pl.BlockSpec((NB, S), lambda i: (i, 0)),
            pl.BlockSpec((D, D), lambda i: (0, 0)),
        ],
        out_specs=[
            pl.BlockSpec((NB, D), lambda i: (i, 0)),
            pl.BlockSpec((NB, TW), lambda i: (i, 0)),
        ],
        out_shape=[
            jax.ShapeDtypeStruct((N, D), jnp.float32),
            jax.ShapeDtypeStruct((N, TW), jnp.float32),
        ],
    )(x_t, x_s, wf0)


def _update_body(out_ref, p0_ref, p1_ref, wf_ref, xs_ref, new_ref, t_ref):
    scat = p0_ref[...] + p1_ref[...]
    new = out_ref[...] + jnp.dot(scat, wf_ref[...],
                                 preferred_element_type=jnp.float32)
    new_ref[...] = new
    t_ref[...] = jnp.concatenate([xs_ref[...], new], axis=1)


def _update_call(out, p0, p1, wf, x_s):
    grid = (N // NB,)
    return pl.pallas_call(
        _update_body,
        grid=grid,
        in_specs=[
            pl.BlockSpec((NB, D), lambda i: (i, 0)),
            pl.BlockSpec((NB, D), lambda i: (i, 0)),
            pl.BlockSpec((NB, D), lambda i: (i, 0)),
            pl.BlockSpec((D, D), lambda i: (0, 0)),
            pl.BlockSpec((NB, S), lambda i: (i, 0)),
        ],
        out_specs=[
            pl.BlockSpec((NB, D), lambda i: (i, 0)),
            pl.BlockSpec((NB, TW), lambda i: (i, 0)),
        ],
        out_shape=[
            jax.ShapeDtypeStruct((N, D), jnp.float32),
            jax.ShapeDtypeStruct((N, TW), jnp.float32),
        ],
    )(out, p0, p1, wf, x_s)


def kernel(x_s, x_t, edge_index, edge_attr, Wf0, Wf1, Wf2, W1, b1, W2, b2):
    f32 = jnp.float32
    eye8 = jnp.eye(8, dtype=f32)
    w1r = jnp.concatenate([W1[0:S], W1[2 * S:2 * S + D]], axis=0)
    w1c = jnp.concatenate([W1[S:2 * S], W1[2 * S + D:2 * S + 2 * D]], axis=0)
    w1e = W1[2 * S + 2 * D:]
    bdr = jnp.kron(eye8, w1r)
    bdc = jnp.kron(eye8, w1c)
    sels = [jnp.kron(eye8, w1e[i].reshape(1, HID)) for i in range(F)]
    b1t = jnp.tile(b1, 8).reshape(1, 128)
    w2pad = jnp.concatenate([jnp.zeros((HID, S), f32), W2], axis=1)
    bdw2 = jnp.kron(eye8, w2pad)
    b2t = jnp.tile(jnp.concatenate([jnp.zeros((S,), f32), b2]), 8)
    b2t = b2t.reshape(1, 128)
    bdon = jnp.kron(eye8, jnp.ones((TW, TW), f32))
    mout = jnp.concatenate(
        [jnp.zeros((S, TW), f32), jnp.ones((D, TW), f32)], axis=0)
    bdms = jnp.kron(eye8, mout)
    eacols = [edge_attr[:, i].reshape(E // 8, 8) for i in range(F)]
    zeros_nt = jnp.zeros((N, D), dtype=f32)

    out, t = _init_call(x_t, x_s, Wf0)
    for wf in (Wf1, Wf2):
        gr, gc = _sc_gather_kernel()(t, edge_index)
        grp = gr.reshape(E // 8, 128)
        gcp = gc.reshape(E // 8, 128)
        shiftp = _mlp_call(grp, gcp, eacols, bdr, bdc, sels, b1t, bdw2, b2t,
                           bdon, bdms)
        shift = shiftp.reshape(E, TW)
        p0, p1 = _sc_scatter_kernel()(shift, edge_index, zeros_nt)
        out, t = _update_call(out, p0, p1, wf, x_s)
    return out
